# baseline (speedup 1.0000x reference)
import numpy as np

import jax
import jax.numpy as jnp
from jax.experimental import pallas as pl
from jax.experimental.pallas import tpu as pltpu
from jax.sharding import Mesh, PartitionSpec as P

_VMEM_BUDGET = (64 * 1024 * 1024 * 3) // 4


def _linear_kernel(x_ref, w_ref, b_ref, o_ref):
    acc = jax.lax.dot_general(
        x_ref[...], w_ref[...],
        dimension_numbers=(((1,), (1,)), ((), ())),
        preferred_element_type=jnp.float32)
    o_ref[...] = (acc + b_ref[...].astype(jnp.float32)).astype(o_ref.dtype)


def _forward_one_core(x, weight, b_row):
    n, k = x.shape
    h = weight.shape[0]
    out_dtype = x.dtype

    tm = min(1024, n)
    grid = (pl.cdiv(n, tm),)

    bytes_accessed = (x.size * x.dtype.itemsize
                      + weight.size * weight.dtype.itemsize
                      + b_row.size * b_row.dtype.itemsize
                      + n * h * jnp.dtype(out_dtype).itemsize)

    return pl.pallas_call(
        _linear_kernel,
        out_shape=jax.ShapeDtypeStruct((n, h), out_dtype),
        grid=grid,
        in_specs=[
            pl.BlockSpec((tm, k), lambda i: (i, 0)),
            pl.BlockSpec((h, k), lambda i: (0, 0)),
            pl.BlockSpec((1, h), lambda i: (0, 0)),
        ],
        out_specs=pl.BlockSpec((tm, h), lambda i: (i, 0)),
        compiler_params=pltpu.CompilerParams(
            dimension_semantics=("arbitrary",),
            vmem_limit_bytes=_VMEM_BUDGET,
        ),
        cost_estimate=pl.CostEstimate(
            flops=2 * n * h * k,
            bytes_accessed=bytes_accessed,
            transcendentals=0),
    )(x, weight, b_row)


def kernel(x, weight, bias):
    n = x.shape[0]
    h = weight.shape[0]
    b_row = bias.reshape(1, h)

    devs = jax.devices()
    n_cores = 2 if (len(devs) >= 2 and n % 16 == 0) else 1
    if n_cores == 1:
        return _forward_one_core(x, weight, b_row)

    mesh = Mesh(np.array(devs[:2]), ("rows",))
    fwd = jax.shard_map(
        _forward_one_core, mesh=mesh,
        in_specs=(P("rows", None), P(None, None), P(None, None)),
        out_specs=P("rows", None),
        check_vma=False)
    return fwd(x, weight, b_row)


# speedup vs baseline: 13.7385x; 13.7385x over previous
import numpy as np

import jax
import jax.numpy as jnp
from jax.experimental import pallas as pl
from jax.experimental.pallas import tpu as pltpu
from jax.sharding import Mesh, PartitionSpec as P

_VMEM_BUDGET = (64 * 1024 * 1024 * 3) // 4


def _linear_kernel(x_ref, w_ref, b_ref, o_ref):
    acc = jax.lax.dot_general(
        x_ref[...], w_ref[...],
        dimension_numbers=(((1,), (1,)), ((), ())),
        preferred_element_type=jnp.float32)
    o_ref[...] = (acc + b_ref[...].astype(jnp.float32)).astype(o_ref.dtype)


def _forward_one_core(x, weight, b_row):
    n, k = x.shape
    h = weight.shape[0]
    out_dtype = x.dtype

    tm = min(2048, n)
    grid = (pl.cdiv(n, tm),)

    bytes_accessed = (x.size * x.dtype.itemsize
                      + weight.size * weight.dtype.itemsize
                      + b_row.size * b_row.dtype.itemsize
                      + n * h * jnp.dtype(out_dtype).itemsize)

    return pl.pallas_call(
        _linear_kernel,
        out_shape=jax.ShapeDtypeStruct((n, h), out_dtype),
        grid=grid,
        in_specs=[
            pl.BlockSpec((tm, k), lambda i: (i, 0)),
            pl.BlockSpec((h, k), lambda i: (0, 0)),
            pl.BlockSpec((1, h), lambda i: (0, 0)),
        ],
        out_specs=pl.BlockSpec((tm, h), lambda i: (i, 0)),
        compiler_params=pltpu.CompilerParams(
            dimension_semantics=("arbitrary",),
            vmem_limit_bytes=_VMEM_BUDGET,
        ),
        cost_estimate=pl.CostEstimate(
            flops=2 * n * h * k,
            bytes_accessed=bytes_accessed,
            transcendentals=0),
    )(x, weight, b_row)


def kernel(x, weight, bias):
    h = weight.shape[0]
    return _forward_one_core(x, weight, bias.reshape(1, h))
